# Initial kernel scaffold; baseline (speedup 1.0000x reference)
import functools

import jax
import jax.numpy as jnp
from jax import lax
from jax.experimental import pallas as pl
from jax.experimental.pallas import tpu as pltpu
from jax.experimental.pallas import tpu_sc as plsc

_NUM_CORES = 2
_NUM_SUBCORES = 16
_NW = _NUM_CORES * _NUM_SUBCORES

_CHUNK = 128
_NBUF = 4


@functools.partial(jax.jit, static_argnums=(2, 3))
def _gather_rows(idx_flat, table, n, h):
    n_per_w = n // _NW
    n_chunks = n_per_w // _CHUNK
    n_outer = n_chunks // _NBUF

    mesh = plsc.VectorSubcoreMesh(core_axis_name="c", subcore_axis_name="s")

    @functools.partial(
        pl.kernel,
        mesh=mesh,
        out_type=jax.ShapeDtypeStruct((n, h), jnp.float32),
        scratch_types=[
            pltpu.VMEM((n_per_w,), jnp.int32),
            pltpu.VMEM((_NBUF, _CHUNK, h), jnp.float32),
            pltpu.SemaphoreType.DMA((_NBUF,)),
        ],
    )
    def body(idx_hbm, table_hbm, out_hbm, idx_v, rows_v, gsem):
        wid = lax.axis_index("s") * _NUM_CORES + lax.axis_index("c")
        base = wid * n_per_w
        pltpu.sync_copy(idx_hbm.at[pl.ds(base, n_per_w)], idx_v)

        def outer(o, carry):
            c0 = o * _NBUF
            for b in range(_NBUF):
                c = c0 + b
                pltpu.async_copy(
                    table_hbm.at[idx_v.at[pl.ds(c * _CHUNK, _CHUNK)]],
                    rows_v.at[b],
                    gsem.at[b],
                )
            for b in range(_NBUF):
                c = c0 + b
                pltpu.make_async_copy(
                    out_hbm.at[pl.ds(0, _CHUNK)],
                    rows_v.at[b],
                    gsem.at[b],
                ).wait()
                pltpu.sync_copy(
                    rows_v.at[b],
                    out_hbm.at[pl.ds(base + c * _CHUNK, _CHUNK)],
                )
            return carry

        lax.fori_loop(0, n_outer, outer, 0)

    return body(idx_flat, table)


def kernel(indices, table):
    b, l = indices.shape
    _, h = table.shape
    n = b * l
    idx_flat = indices.reshape(n).astype(jnp.int32)
    out = _gather_rows(idx_flat, table, n, h)
    return out.reshape(b, l, h)


# speedup vs baseline: 1.8366x; 1.8366x over previous
import functools

import jax
import jax.numpy as jnp
from jax import lax
from jax.experimental import pallas as pl
from jax.experimental.pallas import tpu as pltpu
from jax.experimental.pallas import tpu_sc as plsc

_NUM_CORES = 2
_NUM_SUBCORES = 16
_NW = _NUM_CORES * _NUM_SUBCORES

_CHUNK = 128
_NBUF = 4


@functools.partial(jax.jit, static_argnums=(2, 3))
def _gather_rows(idx_flat, table, n, h):
    n_per_w = n // _NW
    n_chunks = n_per_w // _CHUNK
    n_outer = n_chunks // _NBUF

    mesh = plsc.VectorSubcoreMesh(core_axis_name="c", subcore_axis_name="s")

    @functools.partial(
        pl.kernel,
        mesh=mesh,
        out_type=jax.ShapeDtypeStruct((n, h), jnp.float32),
        compiler_params=pltpu.CompilerParams(use_tc_tiling_on_sc=False),
        scratch_types=[
            pltpu.VMEM((n_per_w,), jnp.int32),
            pltpu.VMEM((_NBUF, _CHUNK, h), jnp.float32),
            pltpu.SemaphoreType.DMA((_NBUF,)),
        ],
    )
    def body(idx_hbm, table_hbm, out_hbm, idx_v, rows_v, gsem):
        wid = lax.axis_index("s") * _NUM_CORES + lax.axis_index("c")
        base = wid * n_per_w
        pltpu.sync_copy(idx_hbm.at[pl.ds(base, n_per_w)], idx_v)

        def outer(o, carry):
            c0 = o * _NBUF
            for b in range(_NBUF):
                c = c0 + b
                pltpu.async_copy(
                    table_hbm.at[idx_v.at[pl.ds(c * _CHUNK, _CHUNK)]],
                    rows_v.at[b],
                    gsem.at[b],
                )
            for b in range(_NBUF):
                c = c0 + b
                pltpu.make_async_copy(
                    out_hbm.at[pl.ds(0, _CHUNK)],
                    rows_v.at[b],
                    gsem.at[b],
                ).wait()
                pltpu.sync_copy(
                    rows_v.at[b],
                    out_hbm.at[pl.ds(base + c * _CHUNK, _CHUNK)],
                )
            return carry

        lax.fori_loop(0, n_outer, outer, 0)

    return body(idx_flat, table)


def kernel(indices, table):
    b, l = indices.shape
    _, h = table.shape
    n = b * l
    idx_flat = indices.reshape(n).astype(jnp.int32)
    out = _gather_rows(idx_flat, table, n, h)
    return out.reshape(b, l, h)


# speedup vs baseline: 1.8751x; 1.0210x over previous
import functools

import jax
import jax.numpy as jnp
from jax import lax
from jax.experimental import pallas as pl
from jax.experimental.pallas import tpu as pltpu
from jax.experimental.pallas import tpu_sc as plsc

_NUM_CORES = 2
_NUM_SUBCORES = 16
_NW = _NUM_CORES * _NUM_SUBCORES

_CHUNK = 256
_NBUF = 4


@functools.partial(jax.jit, static_argnums=(2, 3))
def _gather_rows(idx_flat, table, n, h):
    n_per_w = n // _NW
    n_chunks = n_per_w // _CHUNK
    n_outer = n_chunks // _NBUF

    mesh = plsc.VectorSubcoreMesh(core_axis_name="c", subcore_axis_name="s")

    @functools.partial(
        pl.kernel,
        mesh=mesh,
        out_type=jax.ShapeDtypeStruct((n, h), jnp.float32),
        compiler_params=pltpu.CompilerParams(use_tc_tiling_on_sc=False),
        scratch_types=[
            pltpu.VMEM((n_per_w,), jnp.int32),
            pltpu.VMEM((_NBUF, _CHUNK, h), jnp.float32),
            pltpu.SemaphoreType.DMA((_NBUF,)),
            pltpu.SemaphoreType.DMA((_NBUF,)),
        ],
    )
    def body(idx_hbm, table_hbm, out_hbm, idx_v, rows_v, gsem, wsem):
        wid = lax.axis_index("s") * _NUM_CORES + lax.axis_index("c")
        base = wid * n_per_w
        pltpu.sync_copy(idx_hbm.at[pl.ds(base, n_per_w)], idx_v)

        def fire_gather(c, b):
            pltpu.async_copy(
                table_hbm.at[idx_v.at[pl.ds(c * _CHUNK, _CHUNK)]],
                rows_v.at[b],
                gsem.at[b],
            )

        def wait_gather(b):
            pltpu.make_async_copy(
                out_hbm.at[pl.ds(0, _CHUNK)], rows_v.at[b], gsem.at[b]
            ).wait()

        def fire_writeback(c, b):
            pltpu.async_copy(
                rows_v.at[b],
                out_hbm.at[pl.ds(base + c * _CHUNK, _CHUNK)],
                wsem.at[b],
            )

        def wait_writeback(b):
            pltpu.make_async_copy(
                rows_v.at[b], out_hbm.at[pl.ds(0, _CHUNK)], wsem.at[b]
            ).wait()

        for b in range(_NBUF):
            fire_gather(b, b)

        def outer(g, carry):
            for b in range(_NBUF):
                c = g * _NBUF + b
                wait_gather(b)
                fire_writeback(c, b)

                @pl.when(g < n_outer - 1)
                def _():
                    wait_writeback(b)
                    fire_gather(c + _NBUF, b)

            return carry

        lax.fori_loop(0, n_outer, outer, 0)
        for b in range(_NBUF):
            wait_writeback(b)

    return body(idx_flat, table)


def kernel(indices, table):
    b, l = indices.shape
    _, h = table.shape
    n = b * l
    idx_flat = indices.reshape(n).astype(jnp.int32)
    out = _gather_rows(idx_flat, table, n, h)
    return out.reshape(b, l, h)
